# baseline (speedup 1.0000x reference)
import functools
import jax
import jax.numpy as jnp
from jax import lax
from jax.experimental import pallas as pl
from jax.experimental.pallas import tpu as pltpu
from jax.experimental.pallas import tpu_sc as plsc

_D = 64
_C = 4
_E = 8
_NOISE_EPS = 0.01
_BN = 4096
_SQRT1_2 = 0.7071067811865476

_N_TC = 16384
_NW = 32
_CH = 128


def _softplus_vec(x):
    u = jnp.exp(-jnp.abs(x))
    s = u / (2.0 + u)
    s2 = s * s
    ln1p = 2.0 * s * (1.0 + s2 * (1.0 / 3.0 + s2 * (0.2 + s2 * (1.0 / 7.0 + s2 / 9.0))))
    return jnp.maximum(x, 0.0) + ln1p


def _ncdf_vec(z):
    xa = jnp.abs(z) * _SQRT1_2
    t = 1.0 / (1.0 + 0.3275911 * xa)
    poly = t * (0.254829592 + t * (-0.284496736 + t * (1.421413741
               + t * (-1.453152027 + t * 1.061405429))))
    erfa = 1.0 - poly * jnp.exp(-xa * xa)
    erf = jnp.where(z < 0.0, -erfa, erfa)
    return 0.5 * (1.0 + erf)


def _sc_moe(x, noise, wcat, a2, prelu16, n_tc):
    n_sc = x.shape[0] - n_tc
    tpw = n_sc // _NW
    nch = tpw // _CH
    npair = _CH // 32
    mesh = plsc.VectorSubcoreMesh(core_axis_name="c", subcore_axis_name="s",
                                  num_cores=2, num_subcores=16)

    @functools.partial(
        pl.kernel, mesh=mesh,
        compiler_params=pltpu.CompilerParams(needs_layout_passes=False, use_tc_tiling_on_sc=False),
        out_type=[
            jax.ShapeDtypeStruct((n_sc, _C), jnp.float32),
            jax.ShapeDtypeStruct((_NW, _E, 16), jnp.float32),
            jax.ShapeDtypeStruct((_NW, _E, 16), jnp.float32),
        ],
        scratch_types=[
            pltpu.VMEM((_CH, _D * _C), jnp.float32),
            pltpu.VMEM((_CH, _E), jnp.float32),
            pltpu.VMEM((_CH, _C), jnp.float32),
            pltpu.VMEM((_E, 16), jnp.float32),
            pltpu.VMEM((_E, 16), jnp.float32),
            pltpu.VMEM((_D * _C, 16), jnp.float32),
            pltpu.VMEM((_E, _D), jnp.float32),
            pltpu.VMEM((16,), jnp.float32),
            pltpu.VMEM((_D, 16), jnp.float32),
            pltpu.VMEM((_D * _C * 16, 16), jnp.float32),
            pltpu.VMEM((_D * _E, 16), jnp.float32),
        ],
    )
    def k(x_hbm, nz_hbm, w_hbm, a_hbm, prelu_hbm,
          out_hbm, imp_hbm, load_hbm,
          xbuf, nzbuf, outbuf, impbuf, loadbuf, wv, av, pv, pwt, wsp, pwsp):
        cid = lax.axis_index("c")
        sid = lax.axis_index("s")
        wid = sid * 2 + cid
        base = n_tc + wid * tpw

        iota = lax.iota(jnp.int32, 16)

        pltpu.sync_copy(w_hbm, wv)
        pltpu.sync_copy(a_hbm, av)
        pltpu.sync_copy(prelu_hbm, pv)

        def wsp_body(j, carry):
            wrow = wv[j, :]
            for e in range(16):
                wsp[j * 16 + e, :] = jnp.full((16,), wrow[e], jnp.float32)
            return carry
        lax.fori_loop(0, _D * _C, wsp_body, 0)

        pvec = pv[...]
        for e in range(_E):
            pe = pvec[e]
            for db in range(_D // 16):
                vec = av[e, pl.ds(db * 16, 16)]
                pwv = jnp.where(vec >= 0.0, vec, pe * vec)
                plsc.store_scatter(pwt, [db * 16 + iota,
                                         jnp.full((16,), e, jnp.int32)], pwv)

        def pwsp_body(d, carry):
            prow = pwt[d, :]
            for e in range(_E):
                pwsp[d * _E + e, :] = jnp.full((16,), prow[e], jnp.float32)
            return carry
        lax.fori_loop(0, _D, pwsp_body, 0)

        zrow = jnp.zeros((16,), jnp.float32)
        for e in range(_E):
            impbuf[e, :] = zrow
            loadbuf[e, :] = zrow

        def bf16r(v):
            u = plsc.bitcast(v, jnp.int32)
            u = u + 32767 + ((u >> 16) & 1)
            return plsc.bitcast(u & jnp.int32(-65536), jnp.float32)

        def gate_combine(ridx, accs):
            clean = accs[:_E]
            raw = accs[_E:]
            nz = [plsc.load_gather(nzbuf, [ridx, jnp.full((16,), e, jnp.int32)])
                  for e in range(_E)]
            std = [_softplus_vec(raw[e]) + _NOISE_EPS for e in range(_E)]
            lg = [clean[e] + nz[e] * std[e] for e in range(_E)]

            neg = jnp.float32(-1e30)
            t1 = lg[0]
            for e in range(1, _E):
                t1 = jnp.maximum(t1, lg[e])
            oh1 = []
            taken = None
            for e in range(_E):
                eq = lg[e] == t1
                if taken is None:
                    oh1.append(eq)
                    taken = eq
                else:
                    oh1.append(eq & (~taken))
                    taken = taken | eq
            lg2 = [jnp.where(oh1[e], neg, lg[e]) for e in range(_E)]
            t2 = lg2[0]
            for e in range(1, _E):
                t2 = jnp.maximum(t2, lg2[e])
            oh2 = []
            taken = None
            for e in range(_E):
                eq = lg2[e] == t2
                if taken is None:
                    oh2.append(eq)
                    taken = eq
                else:
                    oh2.append(eq & (~taken))
                    taken = taken | eq
            lg3 = [jnp.where(oh2[e], neg, lg2[e]) for e in range(_E)]
            t3 = lg3[0]
            for e in range(1, _E):
                t3 = jnp.maximum(t3, lg3[e])

            e2 = jnp.exp(t2 - t1)
            denom = 1.0 + e2
            g1 = 1.0 / denom
            g2 = e2 / denom
            zero = jnp.zeros((16,), jnp.float32)
            gates = [jnp.where(oh1[e], g1, zero) + jnp.where(oh2[e], g2, zero)
                     for e in range(_E)]

            for e in range(_E):
                impbuf[e, :] = impbuf[e, :] + gates[e]
                thr = jnp.where(lg[e] > t3, t3, t2)
                prob = _ncdf_vec((clean[e] - thr) / std[e])
                loadbuf[e, :] = loadbuf[e, :] + prob

            def d_body(d, outs):
                gpw = gates[0] * pwsp[d * _E, :]
                for e in range(1, _E):
                    gpw = gpw + gates[e] * pwsp[d * _E + e, :]
                new = []
                for c in range(_C):
                    cidx = jnp.full((16,), d * _C + c, jnp.int32)
                    xv = plsc.load_gather(xbuf, [ridx, cidx])
                    new.append(outs[c] + xv * gpw)
                return tuple(new)

            outs = lax.fori_loop(
                0, _D, d_body,
                tuple(jnp.zeros((16,), jnp.float32) for _ in range(_C)))
            for c in range(_C):
                plsc.store_scatter(outbuf, [ridx, jnp.full((16,), c, jnp.int32)],
                                   outs[c])

        def chunk_body(ch, carry0):
            row0 = base + ch * _CH
            pltpu.sync_copy(x_hbm.at[pl.ds(row0, _CH)], xbuf)
            pltpu.sync_copy(nz_hbm.at[pl.ds(row0, _CH)], nzbuf)

            def pair(p, carry1):
                ridx0 = p * 32 + iota
                ridx1 = ridx0 + 16

                def j_body(j, accs):
                    cidx = jnp.full((16,), j, jnp.int32)
                    xr0 = bf16r(plsc.load_gather(xbuf, [ridx0, cidx]))
                    xr1 = bf16r(plsc.load_gather(xbuf, [ridx1, cidx]))
                    a0 = accs[0]
                    a1 = accs[1]
                    new0 = []
                    new1 = []
                    for e in range(16):
                        we = wsp[j * 16 + e, :]
                        new0.append(a0[e] + xr0 * we)
                        new1.append(a1[e] + xr1 * we)
                    return (tuple(new0), tuple(new1))

                zacc = tuple(jnp.zeros((16,), jnp.float32) for _ in range(16))
                accs0, accs1 = lax.fori_loop(0, _D * _C, j_body, (zacc, zacc))
                gate_combine(ridx0, accs0)
                gate_combine(ridx1, accs1)
                return carry1

            lax.fori_loop(0, npair, pair, 0)

            pltpu.sync_copy(outbuf, out_hbm.at[pl.ds(row0 - n_tc, _CH)])
            return carry0

        lax.fori_loop(0, nch, chunk_body, 0)

        pltpu.sync_copy(impbuf, imp_hbm.at[wid])
        pltpu.sync_copy(loadbuf, load_hbm.at[wid])

    return k(x, noise, wcat, a2, prelu16)


def _moe_block_kernel(x_ref, noise_ref, wg_ref, wn_ref, a_ref, pwv_ref,
                      out_ref, imp_ref, load_ref):
    i = pl.program_id(0)
    bn = x_ref.shape[0]

    x = x_ref[...]
    nz = noise_ref[...]
    clean = jnp.dot(x, wg_ref[...], preferred_element_type=jnp.float32)
    raw = jnp.dot(x, wn_ref[...], preferred_element_type=jnp.float32)
    std = jax.nn.softplus(raw) + _NOISE_EPS
    lg = clean + nz * std

    col = jax.lax.broadcasted_iota(jnp.int32, (bn, _E), 1)
    neg_inf = jnp.float32(-jnp.inf)

    i1 = jnp.argmax(lg, axis=1)[:, None]
    t1 = jnp.max(lg, axis=1, keepdims=True)
    oh1 = col == i1
    lg2 = jnp.where(oh1, neg_inf, lg)
    i2 = jnp.argmax(lg2, axis=1)[:, None]
    t2 = jnp.max(lg2, axis=1, keepdims=True)
    oh2 = col == i2
    lg3 = jnp.where(oh2, neg_inf, lg2)
    t3 = jnp.max(lg3, axis=1, keepdims=True)

    e2 = jnp.exp(t2 - t1)
    denom = 1.0 + e2
    gates = jnp.where(oh1, 1.0 / denom, 0.0) + jnp.where(oh2, e2 / denom, 0.0)

    inv_std = 1.0 / std
    pin = 0.5 * (1.0 + jax.lax.erf((clean - t3) * inv_std * _SQRT1_2))
    pout = 0.5 * (1.0 + jax.lax.erf((clean - t2) * inv_std * _SQRT1_2))
    prob = jnp.where(lg > t3, pin, pout)

    a = a_ref[...]
    pw = jnp.where(a >= 0, a, pwv_ref[...] * a)
    dj = jax.lax.broadcasted_iota(jnp.int32, (_D, _D * _C), 0)
    jj = jax.lax.broadcasted_iota(jnp.int32, (_D, _D * _C), 1)
    rep = (jj // _C == dj).astype(jnp.float32)
    pw_exp = jnp.dot(pw, rep, preferred_element_type=jnp.float32)
    gpw = jnp.dot(gates, pw_exp, preferred_element_type=jnp.float32)
    z = x * gpw
    jc = jax.lax.broadcasted_iota(jnp.int32, (_D * _C, _C), 0)
    cc = jax.lax.broadcasted_iota(jnp.int32, (_D * _C, _C), 1)
    sel = (jc % _C == cc).astype(jnp.float32)
    out_ref[...] = jnp.dot(z, sel, preferred_element_type=jnp.float32)

    @pl.when(i == 0)
    def _init():
        imp_ref[...] = jnp.zeros_like(imp_ref)
        load_ref[...] = jnp.zeros_like(load_ref)

    imp_ref[...] += jnp.sum(gates, axis=0, keepdims=True)
    load_ref[...] += jnp.sum(prob, axis=0, keepdims=True)


def _loss_kernel(tcimp_ref, tcload_ref, scimp_ref, scload_ref, loss_ref):
    rows = scimp_ref.shape[0]
    riota = jax.lax.broadcasted_iota(jnp.int32, (rows, 16), 0) % _E
    liota = jax.lax.broadcasted_iota(jnp.int32, (1, _E), 1)
    m_imp = scimp_ref[...]
    m_load = scload_ref[...]
    tci = tcimp_ref[...]
    tcl = tcload_ref[...]

    def cv2(vals):
        mean = sum(vals) / _E
        var = sum((v - mean) ** 2 for v in vals) / (_E - 1)
        return var / (mean * mean + 1e-10)

    imp = [jnp.sum(jnp.where(riota == e, m_imp, 0.0))
           + jnp.sum(jnp.where(liota == e, tci, 0.0)) for e in range(_E)]
    load = [jnp.sum(jnp.where(riota == e, m_load, 0.0))
            + jnp.sum(jnp.where(liota == e, tcl, 0.0)) for e in range(_E)]
    loss_ref[...] = jnp.broadcast_to(cv2(imp) + cv2(load), (1, 1))


def kernel(muti_int, noise, w_gate, w_noise, a_experts, prelu_w):
    n = muti_int.shape[0]
    x = muti_int.reshape(n, _D * _C)
    a = a_experts.reshape(_E, _D)
    pwv = prelu_w.reshape(_E, 1)
    wcat = jnp.concatenate([w_gate, w_noise], axis=1)

    prelu16 = jnp.pad(prelu_w, (0, 8))
    wcat_r = wcat.astype(jnp.bfloat16).astype(jnp.float32)
    sc_out, sc_imp, sc_load = _sc_moe(x, noise, wcat_r, a, prelu16, _N_TC)

    grid = _N_TC // _BN
    tc_out, tc_imp, tc_load = pl.pallas_call(
        _moe_block_kernel,
        grid=(grid,),
        in_specs=[
            pl.BlockSpec((_BN, _D * _C), lambda i: (i, 0)),
            pl.BlockSpec((_BN, _E), lambda i: (i, 0)),
            pl.BlockSpec((_D * _C, _E), lambda i: (0, 0)),
            pl.BlockSpec((_D * _C, _E), lambda i: (0, 0)),
            pl.BlockSpec((_E, _D), lambda i: (0, 0)),
            pl.BlockSpec((_E, 1), lambda i: (0, 0)),
        ],
        out_specs=[
            pl.BlockSpec((_BN, _C), lambda i: (i, 0)),
            pl.BlockSpec((1, _E), lambda i: (0, 0)),
            pl.BlockSpec((1, _E), lambda i: (0, 0)),
        ],
        out_shape=[
            jax.ShapeDtypeStruct((_N_TC, _C), jnp.float32),
            jax.ShapeDtypeStruct((1, _E), jnp.float32),
            jax.ShapeDtypeStruct((1, _E), jnp.float32),
        ],
    )(x, noise, w_gate, w_noise, a, pwv)

    loss = pl.pallas_call(
        _loss_kernel,
        out_shape=jax.ShapeDtypeStruct((1, 1), jnp.float32),
    )(tc_imp, tc_load,
      sc_imp.reshape(_NW * _E, 16), sc_load.reshape(_NW * _E, 16))

    out = jnp.concatenate([tc_out, sc_out], axis=0)
    return out, loss[0, 0]


# speedup vs baseline: 2.7484x; 2.7484x over previous
import jax
import jax.numpy as jnp
from jax.experimental import pallas as pl

_D = 64
_C = 4
_E = 8
_K = 2
_NOISE_EPS = 0.01
_BN = 4096
_SQRT1_2 = 0.7071067811865476


def _moe_block_kernel(x_ref, noise_ref, wg_ref, wn_ref, a_ref, pwv_ref,
                      out_ref, imp_ref, load_ref, loss_ref):
    i = pl.program_id(0)
    nblocks = pl.num_programs(0)
    bn = x_ref.shape[0]

    x = x_ref[...]
    nz = noise_ref[...]
    clean = jnp.dot(x, wg_ref[...], preferred_element_type=jnp.float32)
    raw = jnp.dot(x, wn_ref[...], preferred_element_type=jnp.float32)
    std = jax.nn.softplus(raw) + _NOISE_EPS
    lg = clean + nz * std

    col = jax.lax.broadcasted_iota(jnp.int32, (bn, _E), 1)
    neg_inf = jnp.float32(-jnp.inf)

    i1 = jnp.argmax(lg, axis=1)[:, None]
    t1 = jnp.max(lg, axis=1, keepdims=True)
    oh1 = col == i1
    lg2 = jnp.where(oh1, neg_inf, lg)
    i2 = jnp.argmax(lg2, axis=1)[:, None]
    t2 = jnp.max(lg2, axis=1, keepdims=True)
    oh2 = col == i2
    lg3 = jnp.where(oh2, neg_inf, lg2)
    t3 = jnp.max(lg3, axis=1, keepdims=True)

    e2 = jnp.exp(t2 - t1)
    denom = 1.0 + e2
    gates = jnp.where(oh1, 1.0 / denom, 0.0) + jnp.where(oh2, e2 / denom, 0.0)

    inv_std = 1.0 / std
    pin = 0.5 * (1.0 + jax.lax.erf((clean - t3) * inv_std * _SQRT1_2))
    pout = 0.5 * (1.0 + jax.lax.erf((clean - t2) * inv_std * _SQRT1_2))
    prob = jnp.where(lg > t3, pin, pout)

    a = a_ref[...]
    pw = jnp.where(a >= 0, a, pwv_ref[...] * a)
    dj = jax.lax.broadcasted_iota(jnp.int32, (_D, _D * _C), 0)
    jj = jax.lax.broadcasted_iota(jnp.int32, (_D, _D * _C), 1)
    rep = (jj // _C == dj).astype(jnp.float32)
    pw_exp = jnp.dot(pw, rep, preferred_element_type=jnp.float32)
    gpw = jnp.dot(gates, pw_exp, preferred_element_type=jnp.float32)
    z = x * gpw
    jc = jax.lax.broadcasted_iota(jnp.int32, (_D * _C, _C), 0)
    cc = jax.lax.broadcasted_iota(jnp.int32, (_D * _C, _C), 1)
    sel = (jc % _C == cc).astype(jnp.float32)
    out_ref[...] = jnp.dot(z, sel, preferred_element_type=jnp.float32)

    @pl.when(i == 0)
    def _init():
        imp_ref[...] = jnp.zeros_like(imp_ref)
        load_ref[...] = jnp.zeros_like(load_ref)
        loss_ref[...] = jnp.zeros_like(loss_ref)

    imp_ref[...] += jnp.sum(gates, axis=0, keepdims=True)
    load_ref[...] += jnp.sum(prob, axis=0, keepdims=True)

    @pl.when(i == nblocks - 1)
    def _finish():
        def cv2(v):
            mean = jnp.sum(v) / _E
            var = jnp.sum((v - mean) ** 2) / (_E - 1)
            return var / (mean * mean + 1e-10)
        val = cv2(imp_ref[0, :]) + cv2(load_ref[0, :])
        loss_ref[...] = jnp.broadcast_to(val, (1, 1))


def kernel(muti_int, noise, w_gate, w_noise, a_experts, prelu_w):
    n = muti_int.shape[0]
    x = muti_int.reshape(n, _D * _C)
    a = a_experts.reshape(_E, _D)
    pwv = prelu_w.reshape(_E, 1)
    grid = n // _BN
    out, _, _, loss = pl.pallas_call(
        _moe_block_kernel,
        grid=(grid,),
        in_specs=[
            pl.BlockSpec((_BN, _D * _C), lambda i: (i, 0)),
            pl.BlockSpec((_BN, _E), lambda i: (i, 0)),
            pl.BlockSpec((_D * _C, _E), lambda i: (0, 0)),
            pl.BlockSpec((_D * _C, _E), lambda i: (0, 0)),
            pl.BlockSpec((_E, _D), lambda i: (0, 0)),
            pl.BlockSpec((_E, 1), lambda i: (0, 0)),
        ],
        out_specs=[
            pl.BlockSpec((_BN, _C), lambda i: (i, 0)),
            pl.BlockSpec((1, _E), lambda i: (0, 0)),
            pl.BlockSpec((1, _E), lambda i: (0, 0)),
            pl.BlockSpec((1, 1), lambda i: (0, 0)),
        ],
        out_shape=[
            jax.ShapeDtypeStruct((n, _C), jnp.float32),
            jax.ShapeDtypeStruct((1, _E), jnp.float32),
            jax.ShapeDtypeStruct((1, _E), jnp.float32),
            jax.ShapeDtypeStruct((1, 1), jnp.float32),
        ],
    )(x, noise, w_gate, w_noise, a, pwv)
    return out, loss[0, 0]
